# baseline (speedup 1.0000x reference)
import jax
import jax.numpy as jnp
from jax.experimental import pallas as pl
from jax.experimental.pallas import tpu as pltpu

_BATCH = 16384
_K = 512
_D = 32
_ALPHA = 0.05
_DAUG = _D + 8
_BLOCK = 8192
_GRID = _BATCH // _BLOCK
_CHUNK = 2048
_NCHUNK = _BLOCK // _CHUNK


def _kmeans_loss_body(et_ref, ct_ref, out_ref, w_ref, eaug_ref, acc_ref):
    i = pl.program_id(0)

    @pl.when(i == 0)
    def _init():
        ct = ct_ref[...]
        w_ref[0:_D, :] = (-2.0 * ct).astype(jnp.bfloat16)
        w_ref[_D:, :] = (
            jnp.broadcast_to(
                jnp.sum(ct * ct, axis=0, keepdims=True), (8, _K)
            ) * jnp.where(
                jax.lax.broadcasted_iota(jnp.int32, (8, _K), 0) == 0, 1.0, 0.0
            )
        ).astype(jnp.bfloat16)
        eaug_ref[_D:, :] = jnp.where(
            jax.lax.broadcasted_iota(jnp.int32, (8, _BLOCK), 0) == 0, 1.0, 0.0
        ).astype(jnp.bfloat16)
        acc_ref[...] = jnp.zeros((8, _CHUNK), jnp.float32)

    eb = et_ref[...].astype(jnp.bfloat16)
    eaug_ref[0:_D, :] = eb

    d2ts = []
    esqs = []
    for c in range(_NCHUNK):
        sl = pl.ds(c * _CHUNK, _CHUNK)
        d2ts.append(jax.lax.dot_general(
            w_ref[...], eaug_ref[:, sl], (((0,), (0,)), ((), ())),
            preferred_element_type=jnp.float32,
        ))
        eb_c = eb[:, c * _CHUNK:(c + 1) * _CHUNK]
        esqs.append(jax.lax.dot_general(
            jnp.ones((_D, 8), jnp.bfloat16), eb_c * eb_c,
            (((0,), (0,)), ((), ())),
            preferred_element_type=jnp.float32,
        ))
    for c in range(_NCHUNK):
        col_min = jnp.min(d2ts[c], axis=0, keepdims=True)
        vals = jnp.sqrt(jnp.maximum(col_min + esqs[c][0:1, :], 0.0))
        acc_ref[0:1, :] += vals

    @pl.when(i == _GRID - 1)
    def _finish():
        out_ref[0, 0] = jnp.sum(acc_ref[0:1, :]) * (_ALPHA / _BATCH)


def kernel(embeddings, centers):
    out = pl.pallas_call(
        _kmeans_loss_body,
        grid=(_GRID,),
        in_specs=[
            pl.BlockSpec((_D, _BLOCK), lambda i: (0, i)),
            pl.BlockSpec((_D, _K), lambda i: (0, 0)),
        ],
        out_specs=pl.BlockSpec(memory_space=pltpu.SMEM),
        out_shape=jax.ShapeDtypeStruct((1, 1), jnp.float32),
        scratch_shapes=[
            pltpu.VMEM((_DAUG, _K), jnp.bfloat16),
            pltpu.VMEM((_DAUG, _BLOCK), jnp.bfloat16),
            pltpu.VMEM((8, _CHUNK), jnp.float32),
        ],
    )(embeddings.T, centers.T)
    return out[0, 0]


# speedup vs baseline: 1.0108x; 1.0108x over previous
import jax
import jax.numpy as jnp
from jax.experimental import pallas as pl
from jax.experimental.pallas import tpu as pltpu

_BATCH = 16384
_K = 512
_D = 32
_ALPHA = 0.05
_DAUG = _D + 8
_BLOCK = 16384
_GRID = _BATCH // _BLOCK
_CHUNK = 2048
_NCHUNK = _BLOCK // _CHUNK


def _kmeans_loss_body(et_ref, ct_ref, out_ref, w_ref, eaug_ref, acc_ref):
    i = pl.program_id(0)

    @pl.when(i == 0)
    def _init():
        ct = ct_ref[...]
        w_ref[0:_D, :] = (-2.0 * ct).astype(jnp.bfloat16)
        w_ref[_D:, :] = (
            jnp.broadcast_to(
                jnp.sum(ct * ct, axis=0, keepdims=True), (8, _K)
            ) * jnp.where(
                jax.lax.broadcasted_iota(jnp.int32, (8, _K), 0) == 0, 1.0, 0.0
            )
        ).astype(jnp.bfloat16)
        eaug_ref[_D:, :] = jnp.where(
            jax.lax.broadcasted_iota(jnp.int32, (8, _BLOCK), 0) == 0, 1.0, 0.0
        ).astype(jnp.bfloat16)
        acc_ref[...] = jnp.zeros((8, _CHUNK), jnp.float32)

    eb = et_ref[...].astype(jnp.bfloat16)
    eaug_ref[0:_D, :] = eb

    d2ts = []
    esqs = []
    for c in range(_NCHUNK):
        sl = pl.ds(c * _CHUNK, _CHUNK)
        d2ts.append(jax.lax.dot_general(
            w_ref[...], eaug_ref[:, sl], (((0,), (0,)), ((), ())),
            preferred_element_type=jnp.float32,
        ))
        eb_c = eb[:, c * _CHUNK:(c + 1) * _CHUNK]
        esqs.append(jax.lax.dot_general(
            jnp.ones((_D, 8), jnp.bfloat16), eb_c * eb_c,
            (((0,), (0,)), ((), ())),
            preferred_element_type=jnp.float32,
        ))
    for c in range(_NCHUNK):
        col_min = jnp.min(d2ts[c], axis=0, keepdims=True)
        vals = jnp.sqrt(jnp.maximum(col_min + esqs[c][0:1, :], 0.0))
        acc_ref[0:1, :] += vals

    @pl.when(i == _GRID - 1)
    def _finish():
        out_ref[0, 0] = jnp.sum(acc_ref[0:1, :]) * (_ALPHA / _BATCH)


def kernel(embeddings, centers):
    out = pl.pallas_call(
        _kmeans_loss_body,
        grid=(_GRID,),
        in_specs=[
            pl.BlockSpec((_D, _BLOCK), lambda i: (0, i)),
            pl.BlockSpec((_D, _K), lambda i: (0, 0)),
        ],
        out_specs=pl.BlockSpec(memory_space=pltpu.SMEM),
        out_shape=jax.ShapeDtypeStruct((1, 1), jnp.float32),
        scratch_shapes=[
            pltpu.VMEM((_DAUG, _K), jnp.bfloat16),
            pltpu.VMEM((_DAUG, _BLOCK), jnp.bfloat16),
            pltpu.VMEM((8, _CHUNK), jnp.float32),
        ],
    )(embeddings.T, centers.T)
    return out[0, 0]
